# baseline (speedup 1.0000x reference)
import functools

import jax
import jax.numpy as jnp
from jax import lax
from jax.experimental import pallas as pl
from jax.experimental.pallas import tpu as pltpu
from jax.experimental.pallas import tpu_sc as plsc

CHUNK_B = 4
NBUF = 2


@functools.partial(jax.jit, static_argnames=("n_chunks", "rows_per_w", "num_cores"))
def _sc_gather(table, idx, *, n_chunks, rows_per_w, num_cores):
    d = table.shape[1]
    nb, t = idx.shape
    mesh = plsc.VectorSubcoreMesh(core_axis_name="c", subcore_axis_name="s")

    splits = []
    off = 0
    while off < t:
        n = min(128, t - off)
        splits.append((off, n))
        off += n

    @functools.partial(
        pl.kernel,
        out_type=jax.ShapeDtypeStruct((nb, t, d), jnp.float32),
        mesh=mesh,
        scratch_types=[
            pltpu.VMEM((NBUF, CHUNK_B, t), jnp.int32),
            pltpu.VMEM((NBUF, CHUNK_B, t, d), jnp.float32),
            pltpu.VMEM_SHARED(table.shape, jnp.float32),
            pltpu.SemaphoreType.DMA((NBUF,)),
            pltpu.SemaphoreType.DMA((NBUF,)),
            pltpu.SemaphoreType.DMA((NBUF,)),
        ],
        compiler_params=pltpu.CompilerParams(use_tc_tiling_on_sc=False),
    )
    def k(table_hbm, idx_hbm, out_hbm, idx_v, rows_v, table_sh, sem_i, sem_g, sem_o):
        wid = lax.axis_index("s") * num_cores + lax.axis_index("c")
        base = wid * rows_per_w

        @pl.when(lax.axis_index("s") == 0)
        def _():
            pltpu.sync_copy(table_hbm, table_sh)

        plsc.subcore_barrier()

        def idx_copy(c, b):
            return pltpu.make_async_copy(
                idx_hbm.at[pl.ds(base + c * CHUNK_B, CHUNK_B), :],
                idx_v.at[b],
                sem_i.at[b],
            )

        def gather_copy(b, j, off, n):
            return pltpu.make_async_copy(
                table_sh.at[idx_v.at[b, j, pl.ds(off, n)]],
                rows_v.at[b, j, pl.ds(off, n), :],
                sem_g.at[b],
            )

        def out_copy(c, b):
            return pltpu.make_async_copy(
                rows_v.at[b],
                out_hbm.at[pl.ds(base + c * CHUNK_B, CHUNK_B), :, :],
                sem_o.at[b],
            )

        idx_copy(0, 0).start()

        def step(i, _):
            c2 = 2 * i
            for b in range(NBUF):
                c = c2 + b

                @pl.when(c + 1 < n_chunks)
                def _():
                    idx_copy(c + 1, b ^ 1).start()

                idx_copy(c, b).wait()

                @pl.when(c2 >= 2)
                def _():
                    out_copy(c - NBUF, b).wait()

                for j in range(CHUNK_B):
                    for off, n in splits:
                        gather_copy(b, j, off, n).start()
                for j in range(CHUNK_B):
                    for off, n in splits:
                        gather_copy(b, j, off, n).wait()

                out_copy(c, b).start()
            return _

        lax.fori_loop(0, n_chunks // NBUF, step, None)

        for b in range(NBUF):
            out_copy(n_chunks - NBUF + b, b).wait()

    return k(table, idx)


def kernel(time, time_encoding):
    nb, t = time.shape

    info = plsc.get_sparse_core_info()
    num_workers = info.num_cores * info.num_subcores
    rows_per_w = nb // num_workers
    assert rows_per_w * num_workers == nb and rows_per_w % CHUNK_B == 0

    return _sc_gather(
        jnp.asarray(time_encoding, jnp.float32),
        jnp.asarray(time, jnp.int32),
        n_chunks=rows_per_w // CHUNK_B,
        rows_per_w=rows_per_w,
        num_cores=info.num_cores,
    )
